# baseline (speedup 1.0000x reference)
import functools

import jax
import jax.numpy as jnp
from jax import lax
from jax.experimental import pallas as pl
from jax.experimental.pallas import tpu as pltpu
from jax.experimental.pallas import tpu_sc as plsc

BATCH = 4096
HIST_LEN = 200
EMBEDDING_DIM = 64

NUM_CORES = 2
NUM_SUBCORES = 16
NUM_WORKERS = NUM_CORES * NUM_SUBCORES
BW = BATCH // NUM_WORKERS
B_PER_W = HIST_LEN * BW
LANES = 16

CHUNK = 2 * BW
N_CHUNKS = B_PER_W // CHUNK
NBUF = 4
LAG = 3

_mesh = plsc.VectorSubcoreMesh(core_axis_name="c", subcore_axis_name="s")


@functools.partial(
    pl.kernel,
    out_type=jax.ShapeDtypeStruct(
        (HIST_LEN, EMBEDDING_DIM // 8, NUM_WORKERS, 8, BW), jnp.float32
    ),
    mesh=_mesh,
    compiler_params=pltpu.CompilerParams(
        use_tc_tiling_on_sc=False,
        disable_bounds_checks=True,
        needs_layout_passes=False,
    ),
    scratch_types=[
        pltpu.VMEM((B_PER_W,), jnp.int32),
        [pltpu.VMEM((CHUNK, EMBEDDING_DIM), jnp.float32) for _ in range(NBUF)],
        [
            pltpu.VMEM((2, EMBEDDING_DIM // 8, 8, BW + 1), jnp.float32)
            for _ in range(2)
        ],
        [pltpu.SemaphoreType.DMA for _ in range(NBUF)],
        [pltpu.SemaphoreType.DMA for _ in range(2)],
    ],
)
def _sc_embed(idx_hbm, table_hbm, out_hbm, idx_v, rows, blks, gsem, wsem):
    wid = lax.axis_index("s") * NUM_CORES + lax.axis_index("c")
    pltpu.sync_copy(idx_hbm.at[wid], idx_v)

    def gather_copy(g, b):
        return pltpu.make_async_copy(
            table_hbm.at[idx_v.at[pl.ds(g * CHUNK, CHUNK)]], rows[b], gsem[b]
        )

    def write_copy(g, b):
        return pltpu.make_async_copy(
            blks[b].at[:, :, :, pl.ds(0, BW)],
            out_hbm.at[pl.ds(2 * g, 2), :, wid],
            wsem[b],
        )

    _iota = lax.iota(jnp.int32, LANES)
    RUNROLL = 8
    _dhi = [(d0 * LANES + _iota) >> 3 for d0 in range(EMBEDDING_DIM // LANES)]
    _dlo = [(d0 * LANES + _iota) & 7 for d0 in range(EMBEDDING_DIM // LANES)]

    def transpose(br, bb):
        def rbody(r0, carry):
            for ru in range(RUNROLL):
                c = r0 * RUNROLL + ru
                cvec = jnp.full((LANES,), c % BW, jnp.int32)
                th = jnp.full((LANES,), c // BW, jnp.int32)
                for d0 in range(EMBEDDING_DIM // LANES):
                    v = rows[br][c, pl.ds(d0 * LANES, LANES)]
                    plsc.store_scatter(
                        blks[bb],
                        [th, _dhi[d0], _dlo[d0], cvec],
                        v,
                    )
            return carry

        lax.fori_loop(0, CHUNK // RUNROLL, rbody, 0)

    gather_copy(0, 0).start()
    gather_copy(1, 1).start()
    gather_copy(2, 2).start()

    def tbody(it, carry):
        for bs in range(NBUF):
            g = it * NBUF + bs
            b2 = bs % 2
            gather_copy(g, bs).wait()

            @pl.when(g + LAG < N_CHUNKS)
            def _():
                gather_copy(g + LAG, (bs + LAG) % NBUF).start()

            @pl.when(g >= 2)
            def _():
                write_copy(g - 2, b2).wait()

            transpose(bs, b2)
            write_copy(g, b2).start()

        return carry

    lax.fori_loop(0, N_CHUNKS // NBUF, tbody, 0)
    write_copy(N_CHUNKS - 2, 0).wait()
    write_copy(N_CHUNKS - 1, 1).wait()


def kernel(token_ids, weights):
    idx_w = (
        token_ids.T.reshape(HIST_LEN, NUM_WORKERS, BW)
        .transpose(1, 0, 2)
        .reshape(NUM_WORKERS, B_PER_W)
    )
    out5 = _sc_embed(idx_w, weights)
    return out5.transpose(2, 4, 0, 1, 3).reshape(BATCH, HIST_LEN, EMBEDDING_DIM)


# speedup vs baseline: 1.0132x; 1.0132x over previous
import functools

import jax
import jax.numpy as jnp
from jax import lax
from jax.experimental import pallas as pl
from jax.experimental.pallas import tpu as pltpu
from jax.experimental.pallas import tpu_sc as plsc

BATCH = 4096
HIST_LEN = 200
EMBEDDING_DIM = 64
B_TOTAL = BATCH * HIST_LEN

NUM_CORES = 2
NUM_SUBCORES = 16
NUM_WORKERS = NUM_CORES * NUM_SUBCORES
BW = BATCH // NUM_WORKERS
LANES = 16

_mesh = plsc.VectorSubcoreMesh(core_axis_name="c", subcore_axis_name="s")


@functools.partial(
    pl.kernel,
    out_type=jax.ShapeDtypeStruct(
        (HIST_LEN, EMBEDDING_DIM // 8, NUM_WORKERS, 8, BW), jnp.float32
    ),
    mesh=_mesh,
    compiler_params=pltpu.CompilerParams(
        use_tc_tiling_on_sc=False,
        disable_bounds_checks=True,
        needs_layout_passes=False,
    ),
    scratch_types=[
        pltpu.VMEM((HIST_LEN, BW), jnp.int32),
        [pltpu.VMEM((BW, EMBEDDING_DIM), jnp.float32) for _ in range(4)],
        [pltpu.VMEM((EMBEDDING_DIM // 8, 8, BW + 1), jnp.float32) for _ in range(2)],
        [pltpu.SemaphoreType.DMA for _ in range(4)],
        [pltpu.SemaphoreType.DMA for _ in range(2)],
    ],
)
def _sc_embed(idx_hbm, table_hbm, out_hbm, idx_v, rows, blks, gsem, wsem):
    wid = lax.axis_index("s") * NUM_CORES + lax.axis_index("c")
    pltpu.sync_copy(idx_hbm.at[:, pl.ds(wid * BW, BW)], idx_v)

    def gather_copy(t, b):
        return pltpu.make_async_copy(
            table_hbm.at[idx_v.at[t]], rows[b], gsem[b]
        )

    def write_copy(t, b):
        return pltpu.make_async_copy(
            blks[b].at[:, :, pl.ds(0, BW)], out_hbm.at[t, :, wid], wsem[b]
        )

    _iota = lax.iota(jnp.int32, LANES)
    RUNROLL = 8

    def transpose(br, bb):
        def rbody(r0, carry):
            for ru in range(RUNROLL):
                c = r0 * RUNROLL + ru
                cvec = jnp.full((LANES,), c, jnp.int32)
                for d0 in range(EMBEDDING_DIM // LANES):
                    v = rows[br][c, pl.ds(d0 * LANES, LANES)]
                    d = d0 * LANES + _iota
                    plsc.store_scatter(
                        blks[bb],
                        [d >> 3, d & 7, cvec],
                        v,
                    )
            return carry

        lax.fori_loop(0, BW // RUNROLL, rbody, 0)

    gather_copy(0, 0).start()
    gather_copy(1, 1).start()

    def tbody(g, carry):
        for bs in range(4):
            t = g * 4 + bs
            b2 = bs % 2
            gather_copy(t, bs).wait()

            @pl.when(t + 2 < HIST_LEN)
            def _():
                gather_copy(t + 2, (bs + 2) % 4).start()

            @pl.when(t >= 2)
            def _():
                write_copy(t - 2, b2).wait()

            transpose(bs, b2)
            write_copy(t, b2).start()

        return carry

    lax.fori_loop(0, HIST_LEN // 4, tbody, 0)
    write_copy(HIST_LEN - 2, 0).wait()
    write_copy(HIST_LEN - 1, 1).wait()


def kernel(token_ids, weights):
    idx_t = token_ids.T
    out5 = _sc_embed(idx_t, weights)
    return out5.transpose(2, 4, 0, 1, 3).reshape(BATCH, HIST_LEN, EMBEDDING_DIM)
